# Initial kernel scaffold; baseline (speedup 1.0000x reference)
import jax
import jax.numpy as jnp
from jax.experimental import pallas as pl


def kernel(x, edge_index, Wl1, Wr1, b1, Wl2, Wr2, b2):
    raise NotImplementedError("write your pallas kernel here")


# speedup vs baseline: 4.4208x; 4.4208x over previous
import functools

import jax
import jax.numpy as jnp
from jax import lax
from jax.experimental import pallas as pl
from jax.experimental.pallas import tpu as pltpu
from jax.experimental.pallas import tpu_sc as plsc

N = 10000
NPAD = 10240
E = 320000
IN_DIM = 128
HID = 256
D1 = 144
DH = 128
B = 80
TPS = 640

_MESH = plsc.VectorSubcoreMesh(core_axis_name="c", subcore_axis_name="s")


def _zero_spmem(zbuf, agg, s, ncols):
    z16 = jnp.zeros((16,), jnp.float32)
    nc16 = ncols // 16

    def zrow(i, _):
        def zcol(j, _):
            zbuf[i, pl.ds(j * 16, 16)] = z16
            return 0
        return lax.fori_loop(0, nc16, zcol, 0)

    lax.fori_loop(0, 40, zrow, 0)

    def zchunk(k, _):
        pltpu.sync_copy(zbuf, agg.at[pl.ds(TPS * s + 40 * k, 40), :])
        return 0

    lax.fori_loop(0, 16, zchunk, 0)


@functools.partial(
    pl.kernel,
    out_type=jax.ShapeDtypeStruct((2, NPAD, D1), jnp.float32),
    mesh=_MESH,
    scratch_types=[
        pltpu.VMEM((40, D1), jnp.float32),
        pltpu.VMEM((B,), jnp.int32),
        pltpu.VMEM((B,), jnp.int32),
        pltpu.VMEM((B, D1), jnp.float32),
        pltpu.VMEM_SHARED((NPAD, D1), jnp.float32),
        pltpu.SemaphoreType.DMA,
    ],
    compiler_params=pltpu.CompilerParams(use_tc_tiling_on_sc=False),
)
def _sc_agg1(xa_hbm, src_hbm, dst_hbm, out_hbm, zbuf, sidx, didx, rows, agg, sem):
    c = lax.axis_index("c")
    s = lax.axis_index("s")
    _zero_spmem(zbuf, agg, s, D1)
    plsc.subcore_barrier()

    ebase = c * (E // 2) + s * (E // 32)

    def step(k, _):
        base = pl.multiple_of(ebase + k * B, 8)
        pltpu.sync_copy(src_hbm.at[pl.ds(base, B)], sidx)
        pltpu.sync_copy(dst_hbm.at[pl.ds(base, B)], didx)
        pltpu.async_copy(xa_hbm.at[sidx], rows, sem).wait()
        pltpu.sync_copy(rows, agg.at[didx], add=True)
        return 0

    lax.fori_loop(0, (E // 32) // B, step, 0)
    plsc.subcore_barrier()

    rbase = TPS * s
    pltpu.sync_copy(agg.at[pl.ds(rbase, TPS), :], out_hbm.at[c, pl.ds(rbase, TPS), :])


@functools.partial(
    pl.kernel,
    out_type=jax.ShapeDtypeStruct((2, NPAD, DH), jnp.float32),
    mesh=_MESH,
    scratch_types=[
        pltpu.VMEM((40, DH), jnp.float32),
        pltpu.VMEM((B,), jnp.int32),
        pltpu.VMEM((B,), jnp.int32),
        pltpu.VMEM((B, DH), jnp.float32),
        pltpu.VMEM_SHARED((NPAD, DH), jnp.float32),
        pltpu.SemaphoreType.DMA,
    ],
    compiler_params=pltpu.CompilerParams(use_tc_tiling_on_sc=False),
)
def _sc_agg2(hs_hbm, src_hbm, dst_hbm, out_hbm, zbuf, sidx, didx, rows, agg, sem):
    c = lax.axis_index("c")
    s = lax.axis_index("s")
    _zero_spmem(zbuf, agg, s, DH)
    plsc.subcore_barrier()

    off = c * N
    ebase = s * (E // 16)

    def step(k, _):
        base = pl.multiple_of(ebase + k * B, 8)
        pltpu.sync_copy(src_hbm.at[pl.ds(base, B)], sidx)

        def addoff(j, _):
            sidx[pl.ds(j * 16, 16)] = sidx[pl.ds(j * 16, 16)] + off
            return 0

        lax.fori_loop(0, B // 16, addoff, 0)
        pltpu.sync_copy(dst_hbm.at[pl.ds(base, B)], didx)
        pltpu.async_copy(hs_hbm.at[sidx], rows, sem).wait()
        pltpu.sync_copy(rows, agg.at[didx], add=True)
        return 0

    lax.fori_loop(0, (E // 16) // B, step, 0)
    plsc.subcore_barrier()

    rbase = TPS * s
    pltpu.sync_copy(agg.at[pl.ds(rbase, TPS), :], out_hbm.at[c, pl.ds(rbase, TPS), :])


_R = 1000


def _dot_t(a, w):
    return lax.dot_general(a, w, (((1,), (1,)), ((), ())),
                           preferred_element_type=jnp.float32)


def _tc1_body(parts_ref, x_ref, wl_ref, wr_ref, b_ref, hs_ref, dinv_ref):
    p0 = parts_ref[0]
    p1 = parts_ref[1]
    agg = p0[:, :IN_DIM] + p1[:, :IN_DIM]
    deg = p0[:, IN_DIM:IN_DIM + 1] + p1[:, IN_DIM:IN_DIM + 1]
    dinv = 1.0 / jnp.maximum(deg, 1.0)
    mean = agg * dinv
    h = _dot_t(mean, wl_ref[...]) + _dot_t(x_ref[...], wr_ref[...]) + b_ref[...][None, :]
    h = jnp.maximum(h, 0.0)
    hs_ref[0] = h[:, :DH]
    hs_ref[1] = h[:, DH:]
    dinv_ref[...] = dinv


def _tc1(parts, x, Wl1, Wr1, b1):
    return pl.pallas_call(
        _tc1_body,
        grid=(N // _R,),
        in_specs=[
            pl.BlockSpec((2, _R, D1), lambda i: (0, i, 0)),
            pl.BlockSpec((_R, IN_DIM), lambda i: (i, 0)),
            pl.BlockSpec((HID, IN_DIM), lambda i: (0, 0)),
            pl.BlockSpec((HID, IN_DIM), lambda i: (0, 0)),
            pl.BlockSpec((HID,), lambda i: (0,)),
        ],
        out_specs=[
            pl.BlockSpec((2, _R, DH), lambda i: (0, i, 0)),
            pl.BlockSpec((_R, 1), lambda i: (i, 0)),
        ],
        out_shape=[
            jax.ShapeDtypeStruct((2, N, DH), jnp.float32),
            jax.ShapeDtypeStruct((N, 1), jnp.float32),
        ],
    )(parts, x, Wl1, Wr1, b1)


def _tc2_body(a2_ref, dinv_ref, hs_ref, wl_ref, wr_ref, b_ref, out_ref):
    dinv = dinv_ref[...]
    m0 = a2_ref[0] * dinv
    m1 = a2_ref[1] * dinv
    acc = (_dot_t(m0, wl_ref[:, :DH]) + _dot_t(m1, wl_ref[:, DH:])
           + _dot_t(hs_ref[0], wr_ref[:, :DH]) + _dot_t(hs_ref[1], wr_ref[:, DH:])
           + b_ref[...][None, :])
    out_ref[...] = jnp.maximum(acc, 0.0)


def _tc2(parts2, dinv, hs, Wl2, Wr2, b2):
    return pl.pallas_call(
        _tc2_body,
        grid=(N // _R,),
        in_specs=[
            pl.BlockSpec((2, _R, DH), lambda i: (0, i, 0)),
            pl.BlockSpec((_R, 1), lambda i: (i, 0)),
            pl.BlockSpec((2, _R, DH), lambda i: (0, i, 0)),
            pl.BlockSpec((HID, HID), lambda i: (0, 0)),
            pl.BlockSpec((HID, HID), lambda i: (0, 0)),
            pl.BlockSpec((HID,), lambda i: (0,)),
        ],
        out_specs=pl.BlockSpec((_R, HID), lambda i: (i, 0)),
        out_shape=jax.ShapeDtypeStruct((N, HID), jnp.float32),
    )(parts2, dinv, hs, Wl2, Wr2, b2)


def kernel(x, edge_index, Wl1, Wr1, b1, Wl2, Wr2, b2):
    src = edge_index[0].astype(jnp.int32)
    dst = edge_index[1].astype(jnp.int32)
    xa = jnp.concatenate(
        [x, jnp.ones((N, 1), jnp.float32), jnp.zeros((N, D1 - IN_DIM - 1), jnp.float32)],
        axis=1)
    parts1 = _sc_agg1(xa, src, dst)
    hs, dinv = _tc1(parts1, x, Wl1, Wr1, b1)
    hflat = hs.reshape(2 * N, DH)
    parts2 = _sc_agg2(hflat, src, dst)
    return _tc2(parts2, dinv, hs, Wl2, Wr2, b2)


# speedup vs baseline: 8.4056x; 1.9014x over previous
import functools

import jax
import jax.numpy as jnp
from jax import lax
from jax.experimental import pallas as pl
from jax.experimental.pallas import tpu as pltpu
from jax.experimental.pallas import tpu_sc as plsc

N = 10000
NPAD = 10240
E = 320000
IN_DIM = 128
HID = 256
D1 = 144
DH = 128
B = 80
TPS = 640

_MESH = plsc.VectorSubcoreMesh(core_axis_name="c", subcore_axis_name="s")


def _zero_spmem(zbuf, agg, s, ncols):
    z16 = jnp.zeros((16,), jnp.float32)
    nc16 = ncols // 16

    def zrow(i, _):
        def zcol(j, _):
            zbuf[i, pl.ds(j * 16, 16)] = z16
            return 0
        return lax.fori_loop(0, nc16, zcol, 0)

    lax.fori_loop(0, 40, zrow, 0)

    def zchunk(k, _):
        pltpu.sync_copy(zbuf, agg.at[pl.ds(TPS * s + 40 * k, 40), :])
        return 0

    lax.fori_loop(0, 16, zchunk, 0)


def _edge_pipeline(ei, tab, agg, ibuf, rows2, semi, semg, ebase, nsteps, off):
    def idx_start(k):
        base = pl.multiple_of(ebase + k * B, 8)
        return pltpu.async_copy(ei.at[:, pl.ds(base, B)], ibuf.at[k % 2], semi)

    def add_off(p):
        if off is None:
            return

        def body(j, _):
            ibuf[p, 0, pl.ds(j * 16, 16)] = ibuf[p, 0, pl.ds(j * 16, 16)] + off
            return 0

        lax.fori_loop(0, B // 16, body, 0)

    def gather_start(p):
        return pltpu.async_copy(tab.at[ibuf.at[p, 0]], rows2.at[p], semg)

    def gather_wait(p):
        pltpu.make_async_copy(tab.at[ibuf.at[p, 0]], rows2.at[p], semg).wait()

    def scatter(p):
        pltpu.sync_copy(rows2.at[p], agg.at[ibuf.at[p, 1]], add=True)

    idx_start(0).wait()
    add_off(0)
    gather_start(0)
    idx_start(1)

    def body(k, _):
        p = k % 2
        pn = (k + 1) % 2
        pltpu.make_async_copy(ei.at[:, pl.ds(0, B)], ibuf.at[pn], semi).wait()
        add_off(pn)
        gather_start(pn)
        gather_wait(p)
        scatter(p)
        idx_start(jnp.minimum(k + 2, nsteps - 1))
        return 0

    lax.fori_loop(0, nsteps - 1, body, 0)

    pe = (nsteps - 1) % 2
    pltpu.make_async_copy(ei.at[:, pl.ds(0, B)], ibuf.at[nsteps % 2], semi).wait()
    gather_wait(pe)
    scatter(pe)


@functools.partial(
    pl.kernel,
    out_type=jax.ShapeDtypeStruct((2, NPAD, D1), jnp.float32),
    mesh=_MESH,
    scratch_types=[
        pltpu.VMEM((40, D1), jnp.float32),
        pltpu.VMEM((2, 2, B), jnp.int32),
        pltpu.VMEM((2, B, D1), jnp.float32),
        pltpu.VMEM_SHARED((NPAD, D1), jnp.float32),
        pltpu.SemaphoreType.DMA,
        pltpu.SemaphoreType.DMA,
    ],
    compiler_params=pltpu.CompilerParams(use_tc_tiling_on_sc=False),
)
def _sc_agg1(xa_hbm, ei_hbm, out_hbm, zbuf, ibuf, rows2, agg, semi, semg):
    c = lax.axis_index("c")
    s = lax.axis_index("s")
    _zero_spmem(zbuf, agg, s, D1)
    plsc.subcore_barrier()

    ebase = c * (E // 2) + s * (E // 32)
    _edge_pipeline(ei_hbm, xa_hbm, agg, ibuf, rows2, semi, semg,
                   ebase, (E // 32) // B, None)
    plsc.subcore_barrier()

    rbase = TPS * s
    pltpu.sync_copy(agg.at[pl.ds(rbase, TPS), :], out_hbm.at[c, pl.ds(rbase, TPS), :])


@functools.partial(
    pl.kernel,
    out_type=jax.ShapeDtypeStruct((2, NPAD, DH), jnp.float32),
    mesh=_MESH,
    scratch_types=[
        pltpu.VMEM((40, DH), jnp.float32),
        pltpu.VMEM((2, 2, B), jnp.int32),
        pltpu.VMEM((2, B, DH), jnp.float32),
        pltpu.VMEM_SHARED((NPAD, DH), jnp.float32),
        pltpu.SemaphoreType.DMA,
        pltpu.SemaphoreType.DMA,
    ],
    compiler_params=pltpu.CompilerParams(use_tc_tiling_on_sc=False),
)
def _sc_agg2(hs_hbm, ei_hbm, out_hbm, zbuf, ibuf, rows2, agg, semi, semg):
    c = lax.axis_index("c")
    s = lax.axis_index("s")
    _zero_spmem(zbuf, agg, s, DH)
    plsc.subcore_barrier()

    ebase = s * (E // 16)
    _edge_pipeline(ei_hbm, hs_hbm, agg, ibuf, rows2, semi, semg,
                   ebase, (E // 16) // B, c * N)
    plsc.subcore_barrier()

    rbase = TPS * s
    pltpu.sync_copy(agg.at[pl.ds(rbase, TPS), :], out_hbm.at[c, pl.ds(rbase, TPS), :])


_R = 1000


def _dot_t(a, w):
    return lax.dot_general(a, w, (((1,), (1,)), ((), ())),
                           preferred_element_type=jnp.float32)


def _tc1_body(parts_ref, x_ref, wl_ref, wr_ref, b_ref, hs_ref, dinv_ref):
    p0 = parts_ref[0]
    p1 = parts_ref[1]
    agg = p0[:, :IN_DIM] + p1[:, :IN_DIM]
    deg = p0[:, IN_DIM:IN_DIM + 1] + p1[:, IN_DIM:IN_DIM + 1]
    dinv = 1.0 / jnp.maximum(deg, 1.0)
    mean = agg * dinv
    h = _dot_t(mean, wl_ref[...]) + _dot_t(x_ref[...], wr_ref[...]) + b_ref[...][None, :]
    h = jnp.maximum(h, 0.0)
    hs_ref[0] = h[:, :DH]
    hs_ref[1] = h[:, DH:]
    dinv_ref[...] = dinv


def _tc1(parts, x, Wl1, Wr1, b1):
    return pl.pallas_call(
        _tc1_body,
        grid=(N // _R,),
        in_specs=[
            pl.BlockSpec((2, _R, D1), lambda i: (0, i, 0)),
            pl.BlockSpec((_R, IN_DIM), lambda i: (i, 0)),
            pl.BlockSpec((HID, IN_DIM), lambda i: (0, 0)),
            pl.BlockSpec((HID, IN_DIM), lambda i: (0, 0)),
            pl.BlockSpec((HID,), lambda i: (0,)),
        ],
        out_specs=[
            pl.BlockSpec((2, _R, DH), lambda i: (0, i, 0)),
            pl.BlockSpec((_R, 1), lambda i: (i, 0)),
        ],
        out_shape=[
            jax.ShapeDtypeStruct((2, N, DH), jnp.float32),
            jax.ShapeDtypeStruct((N, 1), jnp.float32),
        ],
    )(parts, x, Wl1, Wr1, b1)


def _tc2_body(a2_ref, dinv_ref, hs_ref, wl_ref, wr_ref, b_ref, out_ref):
    dinv = dinv_ref[...]
    m0 = a2_ref[0] * dinv
    m1 = a2_ref[1] * dinv
    acc = (_dot_t(m0, wl_ref[:, :DH]) + _dot_t(m1, wl_ref[:, DH:])
           + _dot_t(hs_ref[0], wr_ref[:, :DH]) + _dot_t(hs_ref[1], wr_ref[:, DH:])
           + b_ref[...][None, :])
    out_ref[...] = jnp.maximum(acc, 0.0)


def _tc2(parts2, dinv, hs, Wl2, Wr2, b2):
    return pl.pallas_call(
        _tc2_body,
        grid=(N // _R,),
        in_specs=[
            pl.BlockSpec((2, _R, DH), lambda i: (0, i, 0)),
            pl.BlockSpec((_R, 1), lambda i: (i, 0)),
            pl.BlockSpec((2, _R, DH), lambda i: (0, i, 0)),
            pl.BlockSpec((HID, HID), lambda i: (0, 0)),
            pl.BlockSpec((HID, HID), lambda i: (0, 0)),
            pl.BlockSpec((HID,), lambda i: (0,)),
        ],
        out_specs=pl.BlockSpec((_R, HID), lambda i: (i, 0)),
        out_shape=jax.ShapeDtypeStruct((N, HID), jnp.float32),
    )(parts2, dinv, hs, Wl2, Wr2, b2)


def kernel(x, edge_index, Wl1, Wr1, b1, Wl2, Wr2, b2):
    ei = edge_index.astype(jnp.int32)
    xa = jnp.concatenate(
        [x, jnp.ones((N, 1), jnp.float32), jnp.zeros((N, D1 - IN_DIM - 1), jnp.float32)],
        axis=1)
    parts1 = _sc_agg1(xa, ei)
    hs, dinv = _tc1(parts1, x, Wl1, Wr1, b1)
    hflat = hs.reshape(2 * N, DH)
    parts2 = _sc_agg2(hflat, ei)
    return _tc2(parts2, dinv, hs, Wl2, Wr2, b2)


# speedup vs baseline: 11.9999x; 1.4276x over previous
import functools

import jax
import jax.numpy as jnp
from jax import lax
from jax.experimental import pallas as pl
from jax.experimental.pallas import tpu as pltpu
from jax.experimental.pallas import tpu_sc as plsc

N = 10000
NPAD = 10240
E = 320000
IN_DIM = 128
HID = 256
D1 = 144
DH = 128
B = 80
TPS = 640

_MESH = plsc.VectorSubcoreMesh(core_axis_name="c", subcore_axis_name="s")


def _zero_spmem(zbuf, agg, s, ncols):
    z16 = jnp.zeros((16,), jnp.float32)
    nc16 = ncols // 16

    def zrow(i, _):
        def zcol(j, _):
            zbuf[i, pl.ds(j * 16, 16)] = z16
            return 0
        return lax.fori_loop(0, nc16, zcol, 0)

    lax.fori_loop(0, 16, zrow, 0)

    def zchunk(k, _):
        pltpu.sync_copy(zbuf, agg.at[pl.ds(TPS * s + 16 * k, 16), :])
        return 0

    lax.fori_loop(0, 40, zchunk, 0)


NB = 3
NBI = 8
NSI = 3


def _edge_pipeline(ei, tab, agg, ibuf, rows, semi, semg, sems, ebase, nsteps, off):
    def idx_start(k):
        base = pl.multiple_of(ebase + k * B, 8)
        pltpu.async_copy(ei.at[:, pl.ds(base, B)], ibuf.at[k % NBI],
                         semi.at[k % NSI])

    def idx_wait(k):
        pltpu.make_async_copy(ei.at[:, pl.ds(0, B)], ibuf.at[k % NBI],
                              semi.at[k % NSI]).wait()

    def add_off(k):
        if off is None:
            return
        it = k % NBI

        def body(j, _):
            ibuf[it, 0, pl.ds(j * 16, 16)] = ibuf[it, 0, pl.ds(j * 16, 16)] + off
            return 0

        lax.fori_loop(0, B // 16, body, 0)

    def gather_start(k):
        pltpu.async_copy(tab.at[ibuf.at[k % NBI, 0]], rows.at[k % NB],
                         semg.at[k % 2])

    def gather_wait(k):
        pltpu.make_async_copy(tab.at[ibuf.at[k % NBI, 0]], rows.at[k % NB],
                              semg.at[k % 2]).wait()

    def scatter_start(k):
        pltpu.async_copy(rows.at[k % NB], agg.at[ibuf.at[k % NBI, 1]],
                         sems.at[k % NB], add=True)

    def scatter_wait(k):
        pltpu.make_async_copy(rows.at[k % NB], agg.at[ibuf.at[k % NBI, 1]],
                              sems.at[k % NB]).wait()

    def step(k, do_ws, do_s, do_i):
        idx_wait(k)
        add_off(k)
        gather_start(k)
        if do_s:
            gather_wait(k - 1)
            if do_ws:
                scatter_wait(k - 2)
            scatter_start(k - 1)
        if do_i:
            idx_start(k + NSI)
        return 0

    n = nsteps
    for k in range(NSI):
        idx_start(k)
    step(0, False, False, True)
    step(1, False, True, True)
    lax.fori_loop(2, n - NSI, lambda k, _: step(k, True, True, True), 0)
    for k in range(n - NSI, n):
        step(k, True, True, False)
    gather_wait(n - 1)
    scatter_wait(n - 2)
    scatter_start(n - 1)
    scatter_wait(n - 1)


@functools.partial(
    pl.kernel,
    out_type=jax.ShapeDtypeStruct((2, NPAD, D1), jnp.float32),
    mesh=_MESH,
    scratch_types=[
        pltpu.VMEM((16, D1), jnp.float32),
        pltpu.VMEM((NBI, 2, B), jnp.int32),
        pltpu.VMEM((NB, B, D1), jnp.float32),
        pltpu.VMEM_SHARED((NPAD, D1), jnp.float32),
        pltpu.SemaphoreType.DMA((NSI,)),
        pltpu.SemaphoreType.DMA((2,)),
        pltpu.SemaphoreType.DMA((NB,)),
    ],
    compiler_params=pltpu.CompilerParams(use_tc_tiling_on_sc=False),
)
def _sc_agg1(xa_hbm, ei_hbm, out_hbm, zbuf, ibuf, rows, agg, semi, semg, sems):
    c = lax.axis_index("c")
    s = lax.axis_index("s")
    _zero_spmem(zbuf, agg, s, D1)
    plsc.subcore_barrier()

    ebase = c * (E // 2) + s * (E // 32)
    _edge_pipeline(ei_hbm, xa_hbm, agg, ibuf, rows, semi, semg, sems,
                   ebase, (E // 32) // B, None)
    plsc.subcore_barrier()

    rbase = TPS * s
    pltpu.sync_copy(agg.at[pl.ds(rbase, TPS), :], out_hbm.at[c, pl.ds(rbase, TPS), :])


@functools.partial(
    pl.kernel,
    out_type=jax.ShapeDtypeStruct((2, NPAD, DH), jnp.float32),
    mesh=_MESH,
    scratch_types=[
        pltpu.VMEM((16, DH), jnp.float32),
        pltpu.VMEM((NBI, 2, B), jnp.int32),
        pltpu.VMEM((NB, B, DH), jnp.float32),
        pltpu.VMEM_SHARED((NPAD, DH), jnp.float32),
        pltpu.SemaphoreType.DMA((NSI,)),
        pltpu.SemaphoreType.DMA((2,)),
        pltpu.SemaphoreType.DMA((NB,)),
    ],
    compiler_params=pltpu.CompilerParams(use_tc_tiling_on_sc=False),
)
def _sc_agg2(hs_hbm, ei_hbm, out_hbm, zbuf, ibuf, rows, agg, semi, semg, sems):
    c = lax.axis_index("c")
    s = lax.axis_index("s")
    _zero_spmem(zbuf, agg, s, DH)
    plsc.subcore_barrier()

    ebase = s * (E // 16)
    _edge_pipeline(ei_hbm, hs_hbm, agg, ibuf, rows, semi, semg, sems,
                   ebase, (E // 16) // B, c * N)
    plsc.subcore_barrier()

    rbase = TPS * s
    pltpu.sync_copy(agg.at[pl.ds(rbase, TPS), :], out_hbm.at[c, pl.ds(rbase, TPS), :])


_R = 1000


def _dot_t(a, w):
    return lax.dot_general(a, w, (((1,), (1,)), ((), ())),
                           preferred_element_type=jnp.float32)


def _tc1_body(parts_ref, x_ref, wl_ref, wr_ref, b_ref, hs_ref, dinv_ref):
    p0 = parts_ref[0]
    p1 = parts_ref[1]
    agg = p0[:, :IN_DIM] + p1[:, :IN_DIM]
    deg = p0[:, IN_DIM:IN_DIM + 1] + p1[:, IN_DIM:IN_DIM + 1]
    dinv = 1.0 / jnp.maximum(deg, 1.0)
    mean = agg * dinv
    h = _dot_t(mean, wl_ref[...]) + _dot_t(x_ref[...], wr_ref[...]) + b_ref[...][None, :]
    h = jnp.maximum(h, 0.0)
    hs_ref[0] = h[:, :DH]
    hs_ref[1] = h[:, DH:]
    dinv_ref[...] = dinv


def _tc1(parts, x, Wl1, Wr1, b1):
    return pl.pallas_call(
        _tc1_body,
        grid=(N // _R,),
        in_specs=[
            pl.BlockSpec((2, _R, D1), lambda i: (0, i, 0)),
            pl.BlockSpec((_R, IN_DIM), lambda i: (i, 0)),
            pl.BlockSpec((HID, IN_DIM), lambda i: (0, 0)),
            pl.BlockSpec((HID, IN_DIM), lambda i: (0, 0)),
            pl.BlockSpec((HID,), lambda i: (0,)),
        ],
        out_specs=[
            pl.BlockSpec((2, _R, DH), lambda i: (0, i, 0)),
            pl.BlockSpec((_R, 1), lambda i: (i, 0)),
        ],
        out_shape=[
            jax.ShapeDtypeStruct((2, N, DH), jnp.float32),
            jax.ShapeDtypeStruct((N, 1), jnp.float32),
        ],
    )(parts, x, Wl1, Wr1, b1)


def _tc2_body(a2_ref, dinv_ref, hs_ref, wl_ref, wr_ref, b_ref, out_ref):
    dinv = dinv_ref[...]
    m0 = a2_ref[0] * dinv
    m1 = a2_ref[1] * dinv
    acc = (_dot_t(m0, wl_ref[:, :DH]) + _dot_t(m1, wl_ref[:, DH:])
           + _dot_t(hs_ref[0], wr_ref[:, :DH]) + _dot_t(hs_ref[1], wr_ref[:, DH:])
           + b_ref[...][None, :])
    out_ref[...] = jnp.maximum(acc, 0.0)


def _tc2(parts2, dinv, hs, Wl2, Wr2, b2):
    return pl.pallas_call(
        _tc2_body,
        grid=(N // _R,),
        in_specs=[
            pl.BlockSpec((2, _R, DH), lambda i: (0, i, 0)),
            pl.BlockSpec((_R, 1), lambda i: (i, 0)),
            pl.BlockSpec((2, _R, DH), lambda i: (0, i, 0)),
            pl.BlockSpec((HID, HID), lambda i: (0, 0)),
            pl.BlockSpec((HID, HID), lambda i: (0, 0)),
            pl.BlockSpec((HID,), lambda i: (0,)),
        ],
        out_specs=pl.BlockSpec((_R, HID), lambda i: (i, 0)),
        out_shape=jax.ShapeDtypeStruct((N, HID), jnp.float32),
    )(parts2, dinv, hs, Wl2, Wr2, b2)


def kernel(x, edge_index, Wl1, Wr1, b1, Wl2, Wr2, b2):
    ei = edge_index.astype(jnp.int32)
    xa = jnp.concatenate(
        [x, jnp.ones((N, 1), jnp.float32), jnp.zeros((N, D1 - IN_DIM - 1), jnp.float32)],
        axis=1)
    parts1 = _sc_agg1(xa, ei)
    hs, dinv = _tc1(parts1, x, Wl1, Wr1, b1)
    hflat = hs.reshape(2 * N, DH)
    parts2 = _sc_agg2(hflat, ei)
    return _tc2(parts2, dinv, hs, Wl2, Wr2, b2)
